# baseline (speedup 1.0000x reference)
import functools

import jax
import jax.numpy as jnp
from jax import lax
from jax.experimental import pallas as pl
from jax.experimental.pallas import tpu as pltpu
from jax.experimental.pallas import tpu_sc as plsc

_NL = 9
_L = 11
_START = 9
_END = 10
_LANES = 16
_NC = 2
_NS = 16
_NW = _NC * _NS
_LN2 = 0.6931471805599453


def _log16(s):
    bits = lax.bitcast_convert_type(s, jnp.int32)
    e = lax.shift_right_arithmetic(bits, 23) - 127
    mbits = lax.bitwise_or(lax.bitwise_and(bits, 0x007FFFFF), 0x3F800000)
    m = lax.bitcast_convert_type(mbits, jnp.float32)
    adj = m >= 1.4142135
    m = jnp.where(adj, m * 0.5, m)
    ef = e.astype(jnp.float32) + jnp.where(adj, 1.0, 0.0)
    t = (m - 1.0) / (m + 1.0)
    t2 = t * t
    p = 2.0 * t * (1.0 + t2 * (0.33333333 + t2 * (0.2 + t2 * 0.14285714)))
    return ef * _LN2 + p


def _crf_sc(lgp, lab, lens, tcols, tpad, tend, batch, seq):
    per_w = batch // _NW
    mesh = plsc.VectorSubcoreMesh(core_axis_name="c", subcore_axis_name="s")

    @functools.partial(
        pl.kernel,
        out_type=jax.ShapeDtypeStruct((batch,), jnp.float32),
        mesh=mesh,
        compiler_params=pltpu.CompilerParams(
            use_tc_tiling_on_sc=False, needs_layout_passes=False),
        scratch_types=[
            pltpu.VMEM((per_w * seq * _NL,), jnp.float32),
            pltpu.VMEM((per_w * seq * _NL + _LANES,), jnp.float32),
            pltpu.VMEM((per_w, seq), jnp.int32),
            pltpu.VMEM((_LANES,), jnp.int32),
            pltpu.VMEM((_LANES, _LANES), jnp.float32),
            pltpu.VMEM((_LANES, _LANES), jnp.float32),
            pltpu.VMEM((_LANES,), jnp.float32),
            pltpu.VMEM((_LANES,), jnp.float32),
        ],
    )
    def k(lgp_hbm, lab_hbm, lens_hbm, tcols_hbm, tpad_hbm, tend_hbm,
          out_hbm, lg_v, lgt_v, lab_v, lens_v, tcols_v, tpad_v, tend_v,
          res_v):
        wid = lax.axis_index("s") * _NC + lax.axis_index("c")
        base = wid * per_w
        for j in range(_NL):
            pltpu.sync_copy(
                lgp_hbm.at[pl.ds(j * batch * seq + base * seq, per_w * seq)],
                lg_v.at[pl.ds(j * per_w * seq, per_w * seq)])
        pltpu.sync_copy(lab_hbm.at[pl.ds(base, per_w)], lab_v)
        pltpu.sync_copy(lens_hbm.at[pl.ds(base, per_w)], lens_v.at[pl.ds(0, per_w)])
        pltpu.sync_copy(tcols_hbm, tcols_v)
        pltpu.sync_copy(tpad_hbm, tpad_v)
        pltpu.sync_copy(tend_hbm, tend_v)

        lane = lax.iota(jnp.int32, _LANES)
        lane9x = lane * _NL
        lanef = lane.astype(jnp.float32)
        zero16 = lanef * 0.0
        lens16 = lens_v[:]
        trows = [tcols_v[f, :] for f in range(_L)]
        tendv = tend_v[:]

        def tree(op, xs):
            while len(xs) > 1:
                xs = [op(xs[j], xs[j + 1]) if j + 1 < len(xs) else xs[j]
                      for j in range(0, len(xs), 2)]
            return xs[0]

        dv = tree(jnp.maximum, trows)
        erows = [jnp.exp(trows[f] - dv) for f in range(_L)]
        d_end = jnp.max(tendv, axis=0)
        tendexp = jnp.exp(tendv - d_end)
        emul_pad = jnp.exp(dv - 100.0)
        a_init = jnp.where(lane == _START, 5.184705528587072e21,
                           jnp.where(lane < _L, 1.9287498479639178e-22, 0.0))

        def bcast(vec, idx16):
            return lax.gather(
                vec, idx16[:, None],
                dimension_numbers=lax.GatherDimensionNumbers(
                    offset_dims=(), collapsed_slice_dims=(0,),
                    start_index_map=(0,)),
                slice_sizes=(1,),
                mode=lax.GatherScatterMode.PROMISE_IN_BOUNDS)

        dvj = [bcast(dv, jnp.full((_LANES,), j, jnp.int32))
               for j in range(_NL)]

        res_vec = zero16
        for i in range(per_w):
            ln = lens16[i]
            i16 = jnp.full((_LANES,), i, jnp.int32)

            def reorder(kk, _):
                tb = kk * _LANES
                for j in range(_NL):
                    src = jnp.full((_LANES,), j * (per_w * seq) + i * seq,
                                   jnp.int32) + tb + lane
                    v = plsc.load_gather(lg_v, [src])
                    em = jnp.exp(v + dvj[j])
                    dst = jnp.full((_LANES,), (i * seq + tb) * _NL + j,
                                   jnp.int32) + lane9x
                    plsc.store_scatter(lgt_v, [dst], em)
                return 0

            lax.fori_loop(0, (ln + _LANES - 1) >> 4, reorder, 0)

            def chunk(kk, carry):
                acc_u, acc_b = carry
                t16 = kk * _LANES + lane
                lab_t = plsc.load_gather(lab_v, [i16, t16])
                lab_p = plsc.load_gather(lab_v, [i16, jnp.maximum(t16 - 1, 0)])
                g_u = plsc.load_gather(
                    lg_v, [lab_t * (per_w * seq) + (i * seq) + t16])
                acc_u = acc_u + jnp.where(t16 < ln, g_u, 0.0)
                to16 = jnp.where(t16 < ln, lab_t, _END)
                fr16 = jnp.where(t16 == 0, _START, lab_p)
                g_b = plsc.load_gather(tpad_v, [to16, fr16])
                acc_b = acc_b + jnp.where(t16 <= ln, g_b, 0.0)
                return acc_u, acc_b

            nchunks = lax.shift_right_logical(ln, 4) + 1
            acc_u, acc_b = lax.fori_loop(0, nchunks, chunk, (zero16, zero16))
            gold = jnp.sum(acc_u, axis=0) + jnp.sum(acc_b, axis=0)

            def propagate(av, t):
                g = plsc.load_gather(
                    lgt_v,
                    [jnp.full((_LANES,), (i * seq + t) * _NL, jnp.int32)
                     + lane])
                emul = jnp.where(lane < _NL, g, emul_pad)
                prods = [bcast(av, jnp.full((_LANES,), f, jnp.int32)) * erows[f]
                         for f in range(_L)]
                return tree(jnp.add, prods) * emul

            def renorm(a1, cev):
                sb = bcast(a1, jnp.full((_LANES,), _START, jnp.int32))
                sbits = lax.bitcast_convert_type(sb, jnp.int32)
                ebits = lax.bitwise_and(sbits, 0x7F800000)
                scale = lax.bitcast_convert_type((254 << 23) - ebits,
                                                 jnp.float32)
                cev = cev + lax.shift_right_arithmetic(ebits, 23) - 127
                return a1 * scale, cev

            def dstep(j, carry):
                av, cev = carry
                a1 = propagate(propagate(av, 2 * j), 2 * j + 1)
                return renorm(a1, cev)

            av, cev = lax.fori_loop(0, lax.shift_right_logical(ln, 1),
                                    dstep, (a_init, lane - lane))

            def odd_tail(carry):
                av, cev = carry
                return renorm(propagate(av, ln - 1), cev)

            av, cev = lax.cond(lax.bitwise_and(ln, 1) == 1,
                               odd_tail, lambda c: c, (av, cev))
            ssum = jnp.sum(av * tendexp, axis=0)
            cef = jnp.max(cev, axis=0).astype(jnp.float32)
            norm = (cef * _LN2 + d_end - 50.0 +
                    jnp.max(_log16(jnp.full((_LANES,), ssum)), axis=0))
            res_vec = jnp.where(lane == i, gold - norm, res_vec)

        res_v[:] = res_vec
        pltpu.sync_copy(res_v.at[pl.ds(0, per_w)], out_hbm.at[pl.ds(base, per_w)])

    return k(lgp, lab, lens, tcols, tpad, tend)


def kernel(logits, labels, lens, transition):
    batch, seq, nl = logits.shape
    lgp = jnp.ravel(jnp.transpose(logits.astype(jnp.float32), (2, 0, 1)))
    lab32 = labels.astype(jnp.int32)
    lens32 = lens.astype(jnp.int32)
    t32 = transition.astype(jnp.float32)
    tcols = jnp.full((_LANES, _LANES), -1e30, jnp.float32)
    tcols = tcols.at[:_L, :_L].set(t32.T)
    tpad = jnp.zeros((_LANES, _LANES), jnp.float32).at[:_L, :_L].set(t32)
    tend = jnp.full((_LANES,), -1e30, jnp.float32).at[:_L].set(t32[_END])
    return _crf_sc(lgp, lab32, lens32, tcols, tpad, tend, batch, seq)


# speedup vs baseline: 1.0668x; 1.0668x over previous
import functools

import jax
import jax.numpy as jnp
from jax import lax
from jax.experimental import pallas as pl
from jax.experimental.pallas import tpu as pltpu
from jax.experimental.pallas import tpu_sc as plsc

_NL = 9
_L = 11
_START = 9
_END = 10
_LANES = 16
_NC = 2
_NS = 16
_NW = _NC * _NS
_LN2 = 0.6931471805599453


def _log16(s):
    bits = lax.bitcast_convert_type(s, jnp.int32)
    e = lax.shift_right_arithmetic(bits, 23) - 127
    mbits = lax.bitwise_or(lax.bitwise_and(bits, 0x007FFFFF), 0x3F800000)
    m = lax.bitcast_convert_type(mbits, jnp.float32)
    adj = m >= 1.4142135
    m = jnp.where(adj, m * 0.5, m)
    ef = e.astype(jnp.float32) + jnp.where(adj, 1.0, 0.0)
    t = (m - 1.0) / (m + 1.0)
    t2 = t * t
    p = 2.0 * t * (1.0 + t2 * (0.33333333 + t2 * (0.2 + t2 * 0.14285714)))
    return ef * _LN2 + p


def _crf_sc(lgp, lab, lens, tcols, tpad, tend, batch, seq):
    per_w = batch // _NW
    mesh = plsc.VectorSubcoreMesh(core_axis_name="c", subcore_axis_name="s")

    @functools.partial(
        pl.kernel,
        out_type=jax.ShapeDtypeStruct((batch,), jnp.float32),
        mesh=mesh,
        compiler_params=pltpu.CompilerParams(
            use_tc_tiling_on_sc=False, needs_layout_passes=False),
        scratch_types=[
            pltpu.VMEM((per_w * seq * _NL,), jnp.float32),
            pltpu.VMEM((per_w * seq * _NL + _LANES,), jnp.float32),
            pltpu.VMEM((per_w, seq), jnp.int32),
            pltpu.VMEM((_LANES,), jnp.int32),
            pltpu.VMEM((_LANES, _LANES), jnp.float32),
            pltpu.VMEM((_LANES, _LANES), jnp.float32),
            pltpu.VMEM((_LANES,), jnp.float32),
            pltpu.VMEM((_LANES,), jnp.float32),
            pltpu.SemaphoreType.DMA,
        ],
    )
    def k(lgp_hbm, lab_hbm, lens_hbm, tcols_hbm, tpad_hbm, tend_hbm,
          out_hbm, lg_v, lgt_v, lab_v, lens_v, tcols_v, tpad_v, tend_v,
          res_v, dsem):
        wid = lax.axis_index("s") * _NC + lax.axis_index("c")
        base = wid * per_w
        copies = [
            pltpu.make_async_copy(
                lgp_hbm.at[pl.ds(j * batch * seq + base * seq, per_w * seq)],
                lg_v.at[pl.ds(j * per_w * seq, per_w * seq)], dsem)
            for j in range(_NL)
        ]
        copies += [
            pltpu.make_async_copy(lab_hbm.at[pl.ds(base, per_w)], lab_v, dsem),
            pltpu.make_async_copy(lens_hbm.at[pl.ds(base, per_w)],
                                  lens_v.at[pl.ds(0, per_w)], dsem),
            pltpu.make_async_copy(tcols_hbm, tcols_v, dsem),
            pltpu.make_async_copy(tpad_hbm, tpad_v, dsem),
            pltpu.make_async_copy(tend_hbm, tend_v, dsem),
        ]
        for c in copies:
            c.start()
        for c in copies:
            c.wait()

        lane = lax.iota(jnp.int32, _LANES)
        lane9x = lane * _NL
        lanef = lane.astype(jnp.float32)
        zero16 = lanef * 0.0
        lens16 = lens_v[:]
        trows = [tcols_v[f, :] for f in range(_L)]
        tendv = tend_v[:]

        def tree(op, xs):
            while len(xs) > 1:
                xs = [op(xs[j], xs[j + 1]) if j + 1 < len(xs) else xs[j]
                      for j in range(0, len(xs), 2)]
            return xs[0]

        dv = tree(jnp.maximum, trows)
        erows = [jnp.exp(trows[f] - dv) for f in range(_L)]
        d_end = jnp.max(tendv, axis=0)
        tendexp = jnp.exp(tendv - d_end)
        emul_pad = jnp.exp(dv - 100.0)
        a_init = jnp.where(lane == _START, 5.184705528587072e21,
                           jnp.where(lane < _L, 1.9287498479639178e-22, 0.0))

        def bcast(vec, idx16):
            return lax.gather(
                vec, idx16[:, None],
                dimension_numbers=lax.GatherDimensionNumbers(
                    offset_dims=(), collapsed_slice_dims=(0,),
                    start_index_map=(0,)),
                slice_sizes=(1,),
                mode=lax.GatherScatterMode.PROMISE_IN_BOUNDS)

        dvj = [bcast(dv, jnp.full((_LANES,), j, jnp.int32))
               for j in range(_NL)]

        res_vec = zero16
        for i in range(per_w):
            ln = lens16[i]
            i16 = jnp.full((_LANES,), i, jnp.int32)

            def reorder(kk, _):
                tb = kk * _LANES
                for j in range(_NL):
                    src = jnp.full((_LANES,), j * (per_w * seq) + i * seq,
                                   jnp.int32) + tb + lane
                    v = plsc.load_gather(lg_v, [src])
                    em = jnp.exp(v + dvj[j])
                    dst = jnp.full((_LANES,), (i * seq + tb) * _NL + j,
                                   jnp.int32) + lane9x
                    plsc.store_scatter(lgt_v, [dst], em)
                return 0

            lax.fori_loop(0, (ln + _LANES - 1) >> 4, reorder, 0)

            def chunk(kk, carry):
                acc_u, acc_b = carry
                t16 = kk * _LANES + lane
                lab_t = plsc.load_gather(lab_v, [i16, t16])
                lab_p = plsc.load_gather(lab_v, [i16, jnp.maximum(t16 - 1, 0)])
                g_u = plsc.load_gather(
                    lg_v, [lab_t * (per_w * seq) + (i * seq) + t16])
                acc_u = acc_u + jnp.where(t16 < ln, g_u, 0.0)
                to16 = jnp.where(t16 < ln, lab_t, _END)
                fr16 = jnp.where(t16 == 0, _START, lab_p)
                g_b = plsc.load_gather(tpad_v, [to16, fr16])
                acc_b = acc_b + jnp.where(t16 <= ln, g_b, 0.0)
                return acc_u, acc_b

            nchunks = lax.shift_right_logical(ln, 4) + 1
            acc_u, acc_b = lax.fori_loop(0, nchunks, chunk, (zero16, zero16))
            gold = jnp.sum(acc_u, axis=0) + jnp.sum(acc_b, axis=0)

            def propagate(av, t):
                g = plsc.load_gather(
                    lgt_v,
                    [jnp.full((_LANES,), (i * seq + t) * _NL, jnp.int32)
                     + lane])
                emul = jnp.where(lane < _NL, g, emul_pad)
                prods = [bcast(av, jnp.full((_LANES,), f, jnp.int32)) * erows[f]
                         for f in range(_L)]
                return tree(jnp.add, prods) * emul

            def renorm(a1, cev):
                sb = bcast(a1, jnp.full((_LANES,), _START, jnp.int32))
                sbits = lax.bitcast_convert_type(sb, jnp.int32)
                ebits = lax.bitwise_and(sbits, 0x7F800000)
                scale = lax.bitcast_convert_type((254 << 23) - ebits,
                                                 jnp.float32)
                cev = cev + lax.shift_right_arithmetic(ebits, 23) - 127
                return a1 * scale, cev

            def dstep(j, carry):
                av, cev = carry
                a1 = propagate(propagate(av, 2 * j), 2 * j + 1)
                return renorm(a1, cev)

            av, cev = lax.fori_loop(0, lax.shift_right_logical(ln, 1),
                                    dstep, (a_init, lane - lane))

            def odd_tail(carry):
                av, cev = carry
                return renorm(propagate(av, ln - 1), cev)

            av, cev = lax.cond(lax.bitwise_and(ln, 1) == 1,
                               odd_tail, lambda c: c, (av, cev))
            ssum = jnp.sum(av * tendexp, axis=0)
            cef = jnp.max(cev, axis=0).astype(jnp.float32)
            norm = (cef * _LN2 + d_end - 50.0 +
                    jnp.max(_log16(jnp.full((_LANES,), ssum)), axis=0))
            res_vec = jnp.where(lane == i, gold - norm, res_vec)

        res_v[:] = res_vec
        pltpu.sync_copy(res_v.at[pl.ds(0, per_w)], out_hbm.at[pl.ds(base, per_w)])

    return k(lgp, lab, lens, tcols, tpad, tend)


def kernel(logits, labels, lens, transition):
    batch, seq, nl = logits.shape
    lgp = jnp.ravel(jnp.transpose(logits.astype(jnp.float32), (2, 0, 1)))
    lab32 = labels.astype(jnp.int32)
    lens32 = lens.astype(jnp.int32)
    t32 = transition.astype(jnp.float32)
    tcols = jnp.full((_LANES, _LANES), -1e30, jnp.float32)
    tcols = tcols.at[:_L, :_L].set(t32.T)
    tpad = jnp.zeros((_LANES, _LANES), jnp.float32).at[:_L, :_L].set(t32)
    tend = jnp.full((_LANES,), -1e30, jnp.float32).at[:_L].set(t32[_END])
    return _crf_sc(lgp, lab32, lens32, tcols, tpad, tend, batch, seq)
